# Initial kernel scaffold; baseline (speedup 1.0000x reference)
import jax
import jax.numpy as jnp
from jax.experimental import pallas as pl


def kernel(grid_coords, point_coords, point_feats, W1, b1, W2, b2, gamma, beta):
    raise NotImplementedError("write your pallas kernel here")


# speedup vs baseline: 20.5614x; 20.5614x over previous
import functools

import jax
import jax.numpy as jnp
import numpy as np
from jax import lax
from jax.experimental import pallas as pl
from jax.experimental.pallas import tpu as pltpu
from jax.experimental.pallas import tpu_sc as plsc

_K = 8
_TOPK_MB = 256
_MLP_MC = 512
_ROW_W = 256


def _topk_body(n_points, g_ref, pt_ref, dist_ref, idx_ref):
    b = pl.program_id(0)
    g = g_ref[0]
    pt = pt_ref[0]
    gn = jnp.sum(g * g, axis=1, keepdims=True)
    pn = jnp.sum(pt * pt, axis=0, keepdims=True)
    dgp = lax.dot_general(g, pt, (((1,), (0,)), ((), ())),
                          preferred_element_type=jnp.float32)
    d2 = gn + pn - 2.0 * dgp
    d = jnp.sqrt(jnp.maximum(d2, 1e-12))
    mb, n = d.shape
    iota = lax.broadcasted_iota(jnp.int32, (mb, n), 1)
    base = b * n_points
    for k in range(_K):
        m = jnp.min(d, axis=1, keepdims=True)
        idx = jnp.min(jnp.where(d == m, iota, n), axis=1,
                      keepdims=True)
        dist_ref[0, :, k] = m[:, 0]
        idx_ref[0, :, k] = idx[:, 0] + base
        d = jnp.where(iota == idx, jnp.float32(jnp.inf), d)


def _topk_call(grid_coords, pt_t):
    B, M, _ = grid_coords.shape
    N = pt_t.shape[2]
    grid = (B, M // _TOPK_MB)
    return pl.pallas_call(
        functools.partial(_topk_body, N),
        grid=grid,
        in_specs=[
            pl.BlockSpec((1, _TOPK_MB, 3), lambda b, i: (b, i, 0)),
            pl.BlockSpec((1, 3, N), lambda b, i: (b, 0, 0)),
        ],
        out_specs=[
            pl.BlockSpec((1, _TOPK_MB, _K), lambda b, i: (b, i, 0)),
            pl.BlockSpec((1, _TOPK_MB, _K), lambda b, i: (b, i, 0)),
        ],
        out_shape=[
            jax.ShapeDtypeStruct((B, M, _K), jnp.float32),
            jax.ShapeDtypeStruct((B, M, _K), jnp.int32),
        ],
    )(grid_coords, pt_t)


def _make_gather(n_rows_total, width):
    info = plsc.get_sparse_core_info()
    nw = info.num_cores * info.num_subcores
    per_w = n_rows_total // nw
    ch = 128
    n_ch = per_w // ch
    mesh = plsc.VectorSubcoreMesh(core_axis_name="c", subcore_axis_name="s")

    @functools.partial(
        pl.kernel, mesh=mesh,
        out_type=jax.ShapeDtypeStruct((n_rows_total, width), jnp.float32),
        scratch_types=[
            pltpu.VMEM((ch,), jnp.int32),
            pltpu.VMEM((ch, width), jnp.float32),
            pltpu.SemaphoreType.DMA,
        ],
    )
    def gather_k(table_hbm, idx_hbm, out_hbm, idx_v, rows_v, sem):
        wid = lax.axis_index("s") * info.num_cores + lax.axis_index("c")
        for c in range(n_ch):
            base = wid * per_w + c * ch
            pltpu.sync_copy(idx_hbm.at[pl.ds(base, ch)], idx_v)
            pltpu.async_copy(table_hbm.at[idx_v], rows_v, sem).wait()
            pltpu.sync_copy(rows_v, out_hbm.at[pl.ds(base, ch)])

    return gather_k


def _mlp_body(g_ref, rows_ref, dist_ref, w1_ref, b1_ref, w2_ref, b2_ref,
              gamma_ref, beta_ref, out_ref):
    mc = g_ref.shape[1]
    g = g_ref[0]
    rows = rows_ref[0]
    feats = rows[:, 0:128]
    pc = rows[:, 128:131].reshape(mc, _K, 3)
    w1 = w1_ref[...]
    dot = lambda a, b: lax.dot_general(
        a, b, (((1,), (0,)), ((), ())), preferred_element_type=jnp.float32)
    rel = (g[:, None, :] - pc).reshape(mc * _K, 3)
    pre = dot(rel, w1).reshape(mc, _K, 128) + b1_ref[...][None]
    h = 0.5 * pre * (1.0 + lax.erf(pre * np.float32(1.0 / np.sqrt(2.0))))
    kappa = (dot(h.reshape(mc * _K, 128), w2_ref[...]).reshape(mc, _K, 128)
             + b2_ref[...][None])
    dist = dist_ref[0]
    w = 1.0 / (dist + 1e-6)
    w = w / jnp.sum(w, axis=1, keepdims=True)
    msg = kappa * feats.reshape(mc, _K, 128) * w[:, :, None]
    out = jnp.sum(msg, axis=1)
    mu = jnp.mean(out, axis=1, keepdims=True)
    var = jnp.mean((out - mu) ** 2, axis=1, keepdims=True)
    out = (out - mu) / jnp.sqrt(var + 1e-5)
    out_ref[0] = out * gamma_ref[...] + beta_ref[...]


def _mlp_call(grid_coords, rows, dist, w1, b1, w2, b2, gamma, beta):
    B, M, _ = grid_coords.shape
    grid = (B, M // _MLP_MC)
    return pl.pallas_call(
        _mlp_body,
        grid=grid,
        in_specs=[
            pl.BlockSpec((1, _MLP_MC, 3), lambda b, i: (b, i, 0)),
            pl.BlockSpec((1, _MLP_MC * _K, _ROW_W), lambda b, i: (b, i, 0)),
            pl.BlockSpec((1, _MLP_MC, _K), lambda b, i: (b, i, 0)),
            pl.BlockSpec((3, 128), lambda b, i: (0, 0)),
            pl.BlockSpec((1, 128), lambda b, i: (0, 0)),
            pl.BlockSpec((128, 128), lambda b, i: (0, 0)),
            pl.BlockSpec((1, 128), lambda b, i: (0, 0)),
            pl.BlockSpec((1, 128), lambda b, i: (0, 0)),
            pl.BlockSpec((1, 128), lambda b, i: (0, 0)),
        ],
        out_specs=pl.BlockSpec((1, _MLP_MC, 128), lambda b, i: (b, i, 0)),
        out_shape=jax.ShapeDtypeStruct((B, M, 128), jnp.float32),
    )(grid_coords, rows, dist, w1, b1, w2, b2, gamma, beta)


def kernel(grid_coords, point_coords, point_feats, W1, b1, W2, b2, gamma,
           beta):
    B, M, _ = grid_coords.shape
    N = point_coords.shape[1]
    D = point_feats.shape[2]
    pt_t = jnp.transpose(point_coords, (0, 2, 1))
    dist, idxf = _topk_call(grid_coords, pt_t)
    pad = jnp.zeros((B, N, _ROW_W - D - 3), jnp.float32)
    table = jnp.concatenate([point_feats, point_coords, pad],
                            axis=-1).reshape(B * N, _ROW_W)
    rows = _make_gather(B * M * _K, _ROW_W)(table, idxf.reshape(-1))
    rows = rows.reshape(B, M * _K, _ROW_W)
    return _mlp_call(grid_coords, rows, dist,
                     W1, b1.reshape(1, D), W2, b2.reshape(1, D),
                     gamma.reshape(1, D), beta.reshape(1, D))


# speedup vs baseline: 21.4550x; 1.0435x over previous
import functools

import jax
import jax.numpy as jnp
import numpy as np
from jax import lax
from jax.experimental import pallas as pl
from jax.experimental.pallas import tpu as pltpu
from jax.experimental.pallas import tpu_sc as plsc

_K = 8
_TOPK_MB = 256
_MLP_MC = 512


def _topk_body(n_points, g_ref, pt_ref, dist_ref, idx_ref):
    b = pl.program_id(0)
    g = g_ref[0]
    pt = pt_ref[0]
    gn = jnp.sum(g * g, axis=1, keepdims=True)
    pn = jnp.sum(pt * pt, axis=0, keepdims=True)
    dgp = lax.dot_general(g, pt, (((1,), (0,)), ((), ())),
                          preferred_element_type=jnp.float32)
    d2 = gn + pn - 2.0 * dgp
    d = jnp.sqrt(jnp.maximum(d2, 1e-12))
    mb, n = d.shape
    iota = lax.broadcasted_iota(jnp.int32, (mb, n), 1)
    base = b * n_points
    for k in range(_K):
        m = jnp.min(d, axis=1, keepdims=True)
        idx = jnp.min(jnp.where(d == m, iota, n), axis=1,
                      keepdims=True)
        dist_ref[0, :, k] = m[:, 0]
        idx_ref[0, :, k] = idx[:, 0] + base
        d = jnp.where(iota == idx, jnp.float32(jnp.inf), d)


def _topk_call(grid_coords, pt_t):
    B, M, _ = grid_coords.shape
    N = pt_t.shape[2]
    grid = (B, M // _TOPK_MB)
    return pl.pallas_call(
        functools.partial(_topk_body, N),
        grid=grid,
        in_specs=[
            pl.BlockSpec((1, _TOPK_MB, 3), lambda b, i: (b, i, 0)),
            pl.BlockSpec((1, 3, N), lambda b, i: (b, 0, 0)),
        ],
        out_specs=[
            pl.BlockSpec((1, _TOPK_MB, _K), lambda b, i: (b, i, 0)),
            pl.BlockSpec((1, _TOPK_MB, _K), lambda b, i: (b, i, 0)),
        ],
        out_shape=[
            jax.ShapeDtypeStruct((B, M, _K), jnp.float32),
            jax.ShapeDtypeStruct((B, M, _K), jnp.int32),
        ],
    )(grid_coords, pt_t)


def _make_gather(n_rows_total, width):
    info = plsc.get_sparse_core_info()
    nw = info.num_cores * info.num_subcores
    per_w = n_rows_total // nw
    ch = 128
    n_ch = per_w // ch
    mesh = plsc.VectorSubcoreMesh(core_axis_name="c", subcore_axis_name="s")

    @functools.partial(
        pl.kernel, mesh=mesh,
        out_type=[
            jax.ShapeDtypeStruct((n_rows_total, width), jnp.float32),
            jax.ShapeDtypeStruct((n_rows_total, width), jnp.float32),
        ],
        scratch_types=(
            [pltpu.VMEM((ch,), jnp.int32)] * 2
            + [pltpu.VMEM((ch, width), jnp.float32)] * 4
            + [pltpu.SemaphoreType.DMA] * 8
        ),
    )
    def gather_k(ft_hbm, ct_hbm, idx_hbm, outf_hbm, outc_hbm,
                 i0, i1, f0, f1, c0, c1,
                 sgf0, sgf1, sgc0, sgc1, swf0, swf1, swc0, swc1):
        wid = lax.axis_index("s") * info.num_cores + lax.axis_index("c")
        idxb, fb, cb = [i0, i1], [f0, f1], [c0, c1]
        sgf, sgc = [sgf0, sgf1], [sgc0, sgc1]
        swf, swc = [swf0, swf1], [swc0, swc1]
        gops = [None, None]
        wops = [None, None]
        for c in range(n_ch):
            s = c % 2
            if wops[s] is not None:
                for op in wops[s]:
                    op.wait()
                wops[s] = None
            base = wid * per_w + c * ch
            pltpu.sync_copy(idx_hbm.at[pl.ds(base, ch)], idxb[s])
            gops[s] = (
                pltpu.async_copy(ft_hbm.at[idxb[s]], fb[s], sgf[s]),
                pltpu.async_copy(ct_hbm.at[idxb[s]], cb[s], sgc[s]),
                base,
            )
            s1 = 1 - s
            if gops[s1] is not None:
                gf1, gc1, b1 = gops[s1]
                gf1.wait()
                gc1.wait()
                wops[s1] = (
                    pltpu.async_copy(fb[s1], outf_hbm.at[pl.ds(b1, ch)],
                                     swf[s1]),
                    pltpu.async_copy(cb[s1], outc_hbm.at[pl.ds(b1, ch)],
                                     swc[s1]),
                )
                gops[s1] = None
        s = (n_ch - 1) % 2
        gf, gc, b1 = gops[s]
        gf.wait()
        gc.wait()
        wops[s] = (
            pltpu.async_copy(fb[s], outf_hbm.at[pl.ds(b1, ch)], swf[s]),
            pltpu.async_copy(cb[s], outc_hbm.at[pl.ds(b1, ch)], swc[s]),
        )
        for s in (0, 1):
            if wops[s] is not None:
                for op in wops[s]:
                    op.wait()

    return gather_k


def _mlp_body(g_ref, rowsf_ref, rowsc_ref, dist_ref, w1_ref, b1_ref, w2_ref,
              b2_ref, gamma_ref, beta_ref, out_ref):
    mc = g_ref.shape[1]
    g = g_ref[0]
    feats = rowsf_ref[0]
    pc = rowsc_ref[0][:, 0:3].reshape(mc, _K, 3)
    w1 = w1_ref[...]
    dot = lambda a, b: lax.dot_general(
        a, b, (((1,), (0,)), ((), ())), preferred_element_type=jnp.float32)
    rel = (g[:, None, :] - pc).reshape(mc * _K, 3)
    pre = dot(rel, w1).reshape(mc, _K, 128) + b1_ref[...][None]
    h = 0.5 * pre * (1.0 + lax.erf(pre * np.float32(1.0 / np.sqrt(2.0))))
    kappa = (dot(h.reshape(mc * _K, 128), w2_ref[...]).reshape(mc, _K, 128)
             + b2_ref[...][None])
    dist = dist_ref[0]
    w = 1.0 / (dist + 1e-6)
    w = w / jnp.sum(w, axis=1, keepdims=True)
    msg = kappa * feats.reshape(mc, _K, 128) * w[:, :, None]
    out = jnp.sum(msg, axis=1)
    mu = jnp.mean(out, axis=1, keepdims=True)
    var = jnp.mean((out - mu) ** 2, axis=1, keepdims=True)
    out = (out - mu) / jnp.sqrt(var + 1e-5)
    out_ref[0] = out * gamma_ref[...] + beta_ref[...]


def _mlp_call(grid_coords, rowsf, rowsc, dist, w1, b1, w2, b2, gamma, beta):
    B, M, _ = grid_coords.shape
    grid = (B, M // _MLP_MC)
    return pl.pallas_call(
        _mlp_body,
        grid=grid,
        in_specs=[
            pl.BlockSpec((1, _MLP_MC, 3), lambda b, i: (b, i, 0)),
            pl.BlockSpec((1, _MLP_MC * _K, 128), lambda b, i: (b, i, 0)),
            pl.BlockSpec((1, _MLP_MC * _K, 128), lambda b, i: (b, i, 0)),
            pl.BlockSpec((1, _MLP_MC, _K), lambda b, i: (b, i, 0)),
            pl.BlockSpec((3, 128), lambda b, i: (0, 0)),
            pl.BlockSpec((1, 128), lambda b, i: (0, 0)),
            pl.BlockSpec((128, 128), lambda b, i: (0, 0)),
            pl.BlockSpec((1, 128), lambda b, i: (0, 0)),
            pl.BlockSpec((1, 128), lambda b, i: (0, 0)),
            pl.BlockSpec((1, 128), lambda b, i: (0, 0)),
        ],
        out_specs=pl.BlockSpec((1, _MLP_MC, 128), lambda b, i: (b, i, 0)),
        out_shape=jax.ShapeDtypeStruct((B, M, 128), jnp.float32),
    )(grid_coords, rowsf, rowsc, dist, w1, b1, w2, b2, gamma, beta)


def kernel(grid_coords, point_coords, point_feats, W1, b1, W2, b2, gamma,
           beta):
    B, M, _ = grid_coords.shape
    N = point_coords.shape[1]
    D = point_feats.shape[2]
    pt_t = jnp.transpose(point_coords, (0, 2, 1))
    dist, idxf = _topk_call(grid_coords, pt_t)
    ft = point_feats.reshape(B * N, D)
    ct = jnp.concatenate(
        [point_coords, jnp.zeros((B, N, D - 3), jnp.float32)],
        axis=-1).reshape(B * N, D)
    rowsf, rowsc = _make_gather(B * M * _K, D)(ft, ct, idxf.reshape(-1))
    rowsf = rowsf.reshape(B, M * _K, D)
    rowsc = rowsc.reshape(B, M * _K, D)
    return _mlp_call(grid_coords, rowsf, rowsc, dist,
                     W1, b1.reshape(1, D), W2, b2.reshape(1, D),
                     gamma.reshape(1, D), beta.reshape(1, D))


# speedup vs baseline: 24.7617x; 1.1541x over previous
import functools

import jax
import jax.numpy as jnp
import numpy as np
from jax import lax
from jax.experimental import pallas as pl
from jax.experimental.pallas import tpu as pltpu
from jax.experimental.pallas import tpu_sc as plsc

_K = 8
_TOPK_MB = 256
_MLP_MC = 512


def _topk_body(n_points, g_ref, pt_ref, dist_ref, idx_ref):
    b = pl.program_id(0)
    g = g_ref[0]
    pt = pt_ref[0]
    gn = jnp.sum(g * g, axis=1, keepdims=True)
    pn = jnp.sum(pt * pt, axis=0, keepdims=True)
    dgp = lax.dot_general(g, pt, (((1,), (0,)), ((), ())),
                          preferred_element_type=jnp.float32)
    d2 = gn + pn - 2.0 * dgp
    d = jnp.sqrt(jnp.maximum(d2, 1e-12))
    mb, n = d.shape
    iota = lax.broadcasted_iota(jnp.int32, (mb, n), 1).astype(jnp.float32)
    base = b * n_points
    nf = jnp.float32(n)
    for k in range(_K):
        m = jnp.min(d, axis=1, keepdims=True)
        idx = jnp.min(jnp.where(d == m, iota, nf), axis=1,
                      keepdims=True)
        dist_ref[0, :, k] = m[:, 0]
        idx_ref[0, :, k] = idx[:, 0].astype(jnp.int32) + base
        d = jnp.where(iota == idx, jnp.float32(jnp.inf), d)


def _topk_call(grid_coords, pt_t):
    B, M, _ = grid_coords.shape
    N = pt_t.shape[2]
    grid = (B, M // _TOPK_MB)
    return pl.pallas_call(
        functools.partial(_topk_body, N),
        grid=grid,
        in_specs=[
            pl.BlockSpec((1, _TOPK_MB, 3), lambda b, i: (b, i, 0)),
            pl.BlockSpec((1, 3, N), lambda b, i: (b, 0, 0)),
        ],
        out_specs=[
            pl.BlockSpec((1, _TOPK_MB, _K), lambda b, i: (b, i, 0)),
            pl.BlockSpec((1, _TOPK_MB, _K), lambda b, i: (b, i, 0)),
        ],
        out_shape=[
            jax.ShapeDtypeStruct((B, M, _K), jnp.float32),
            jax.ShapeDtypeStruct((B, M, _K), jnp.int32),
        ],
    )(grid_coords, pt_t)


def _make_gather(n_rows_total, width):
    info = plsc.get_sparse_core_info()
    nw = info.num_cores * info.num_subcores
    per_w = n_rows_total // nw
    ch = 128
    n_ch = per_w // ch
    mesh = plsc.VectorSubcoreMesh(core_axis_name="c", subcore_axis_name="s")

    @functools.partial(
        pl.kernel, mesh=mesh,
        out_type=[
            jax.ShapeDtypeStruct((n_rows_total, width), jnp.float32),
            jax.ShapeDtypeStruct((n_rows_total, width), jnp.float32),
        ],
        scratch_types=(
            [pltpu.VMEM((ch,), jnp.int32)] * 2
            + [pltpu.VMEM((ch, width), jnp.float32)] * 4
            + [pltpu.SemaphoreType.DMA] * 8
        ),
    )
    def gather_k(ft_hbm, ct_hbm, idx_hbm, outf_hbm, outc_hbm,
                 i0, i1, f0, f1, c0, c1,
                 sgf0, sgf1, sgc0, sgc1, swf0, swf1, swc0, swc1):
        wid = lax.axis_index("s") * info.num_cores + lax.axis_index("c")
        idxb, fb, cb = [i0, i1], [f0, f1], [c0, c1]
        sgf, sgc = [sgf0, sgf1], [sgc0, sgc1]
        swf, swc = [swf0, swf1], [swc0, swc1]
        gops = [None, None]
        wops = [None, None]
        for c in range(n_ch):
            s = c % 2
            if wops[s] is not None:
                for op in wops[s]:
                    op.wait()
                wops[s] = None
            base = wid * per_w + c * ch
            pltpu.sync_copy(idx_hbm.at[pl.ds(base, ch)], idxb[s])
            gops[s] = (
                pltpu.async_copy(ft_hbm.at[idxb[s]], fb[s], sgf[s]),
                pltpu.async_copy(ct_hbm.at[idxb[s]], cb[s], sgc[s]),
                base,
            )
            s1 = 1 - s
            if gops[s1] is not None:
                gf1, gc1, b1 = gops[s1]
                gf1.wait()
                gc1.wait()
                wops[s1] = (
                    pltpu.async_copy(fb[s1], outf_hbm.at[pl.ds(b1, ch)],
                                     swf[s1]),
                    pltpu.async_copy(cb[s1], outc_hbm.at[pl.ds(b1, ch)],
                                     swc[s1]),
                )
                gops[s1] = None
        s = (n_ch - 1) % 2
        gf, gc, b1 = gops[s]
        gf.wait()
        gc.wait()
        wops[s] = (
            pltpu.async_copy(fb[s], outf_hbm.at[pl.ds(b1, ch)], swf[s]),
            pltpu.async_copy(cb[s], outc_hbm.at[pl.ds(b1, ch)], swc[s]),
        )
        for s in (0, 1):
            if wops[s] is not None:
                for op in wops[s]:
                    op.wait()

    return gather_k


def _mlp_body(g_ref, rowsf_ref, rowsc_ref, dist_ref, w1_ref, b1_ref, w2_ref,
              b2_ref, gamma_ref, beta_ref, out_ref):
    mc = g_ref.shape[1]
    g = g_ref[0]
    feats = rowsf_ref[0]
    pc = rowsc_ref[0][:, 0:3].reshape(mc, _K, 3)
    w1 = w1_ref[...]
    dot = lambda a, b: lax.dot_general(
        a, b, (((1,), (0,)), ((), ())), preferred_element_type=jnp.float32)
    rel = (g[:, None, :] - pc).reshape(mc * _K, 3)
    pre = dot(rel, w1).reshape(mc, _K, 128) + b1_ref[...][None]
    h = 0.5 * pre * (1.0 + lax.erf(pre * np.float32(1.0 / np.sqrt(2.0))))
    kappa = (dot(h.reshape(mc * _K, 128), w2_ref[...]).reshape(mc, _K, 128)
             + b2_ref[...][None])
    dist = dist_ref[0]
    w = 1.0 / (dist + 1e-6)
    w = w / jnp.sum(w, axis=1, keepdims=True)
    msg = kappa * feats.reshape(mc, _K, 128) * w[:, :, None]
    out = jnp.sum(msg, axis=1)
    mu = jnp.mean(out, axis=1, keepdims=True)
    var = jnp.mean((out - mu) ** 2, axis=1, keepdims=True)
    out = (out - mu) / jnp.sqrt(var + 1e-5)
    out_ref[0] = out * gamma_ref[...] + beta_ref[...]


def _mlp_call(grid_coords, rowsf, rowsc, dist, w1, b1, w2, b2, gamma, beta):
    B, M, _ = grid_coords.shape
    grid = (B, M // _MLP_MC)
    return pl.pallas_call(
        _mlp_body,
        grid=grid,
        in_specs=[
            pl.BlockSpec((1, _MLP_MC, 3), lambda b, i: (b, i, 0)),
            pl.BlockSpec((1, _MLP_MC * _K, 128), lambda b, i: (b, i, 0)),
            pl.BlockSpec((1, _MLP_MC * _K, 128), lambda b, i: (b, i, 0)),
            pl.BlockSpec((1, _MLP_MC, _K), lambda b, i: (b, i, 0)),
            pl.BlockSpec((3, 128), lambda b, i: (0, 0)),
            pl.BlockSpec((1, 128), lambda b, i: (0, 0)),
            pl.BlockSpec((128, 128), lambda b, i: (0, 0)),
            pl.BlockSpec((1, 128), lambda b, i: (0, 0)),
            pl.BlockSpec((1, 128), lambda b, i: (0, 0)),
            pl.BlockSpec((1, 128), lambda b, i: (0, 0)),
        ],
        out_specs=pl.BlockSpec((1, _MLP_MC, 128), lambda b, i: (b, i, 0)),
        out_shape=jax.ShapeDtypeStruct((B, M, 128), jnp.float32),
    )(grid_coords, rowsf, rowsc, dist, w1, b1, w2, b2, gamma, beta)


def kernel(grid_coords, point_coords, point_feats, W1, b1, W2, b2, gamma,
           beta):
    B, M, _ = grid_coords.shape
    N = point_coords.shape[1]
    D = point_feats.shape[2]
    pt_t = jnp.transpose(point_coords, (0, 2, 1))
    dist, idxf = _topk_call(grid_coords, pt_t)
    ft = point_feats.reshape(B * N, D)
    ct = jnp.concatenate(
        [point_coords, jnp.zeros((B, N, D - 3), jnp.float32)],
        axis=-1).reshape(B * N, D)
    rowsf, rowsc = _make_gather(B * M * _K, D)(ft, ct, idxf.reshape(-1))
    rowsf = rowsf.reshape(B, M * _K, D)
    rowsc = rowsc.reshape(B, M * _K, D)
    return _mlp_call(grid_coords, rowsf, rowsc, dist,
                     W1, b1.reshape(1, D), W2, b2.reshape(1, D),
                     gamma.reshape(1, D), beta.reshape(1, D))


# speedup vs baseline: 26.3652x; 1.0648x over previous
import functools

import jax
import jax.numpy as jnp
import numpy as np
from jax import lax
from jax.experimental import pallas as pl
from jax.experimental.pallas import tpu as pltpu
from jax.experimental.pallas import tpu_sc as plsc

_K = 8
_TOPK_MB = 256
_MLP_MC = 512


def _topk_body(n_points, g_ref, pt_ref, dist_ref, idx_ref):
    b = pl.program_id(0)
    g = g_ref[0]
    pt = pt_ref[0]
    gn = jnp.sum(g * g, axis=1, keepdims=True)
    pn = jnp.sum(pt * pt, axis=0, keepdims=True)
    dgp = lax.dot_general(g, pt, (((1,), (0,)), ((), ())),
                          preferred_element_type=jnp.float32)
    d2 = gn + pn - 2.0 * dgp
    d = jnp.sqrt(jnp.maximum(d2, 1e-12))
    mb, n = d.shape
    iota = lax.broadcasted_iota(jnp.int32, (mb, n), 1).astype(jnp.float32)
    base = b * n_points
    nf = jnp.float32(n)
    for k in range(_K):
        m = jnp.min(d, axis=1, keepdims=True)
        idx = jnp.min(jnp.where(d == m, iota, nf), axis=1,
                      keepdims=True)
        dist_ref[0, :, k] = m[:, 0]
        idx_ref[0, :, k] = idx[:, 0].astype(jnp.int32) + base
        d = jnp.where(iota == idx, jnp.float32(jnp.inf), d)


def _topk_call(grid_coords, pt_t):
    B, M, _ = grid_coords.shape
    N = pt_t.shape[2]
    grid = (B, M // _TOPK_MB)
    return pl.pallas_call(
        functools.partial(_topk_body, N),
        grid=grid,
        in_specs=[
            pl.BlockSpec((1, _TOPK_MB, 3), lambda b, i: (b, i, 0)),
            pl.BlockSpec((1, 3, N), lambda b, i: (b, 0, 0)),
        ],
        out_specs=[
            pl.BlockSpec((1, _TOPK_MB, _K), lambda b, i: (b, i, 0)),
            pl.BlockSpec((1, _TOPK_MB, _K), lambda b, i: (b, i, 0)),
        ],
        out_shape=[
            jax.ShapeDtypeStruct((B, M, _K), jnp.float32),
            jax.ShapeDtypeStruct((B, M, _K), jnp.int32),
        ],
    )(grid_coords, pt_t)


def _make_gather(n_rows_total, width):
    info = plsc.get_sparse_core_info()
    nw = info.num_cores * info.num_subcores
    per_w = n_rows_total // nw
    ch = 128
    n_ch = per_w // ch
    mesh = plsc.VectorSubcoreMesh(core_axis_name="c", subcore_axis_name="s")

    @functools.partial(
        pl.kernel, mesh=mesh,
        out_type=[
            jax.ShapeDtypeStruct((n_rows_total, width), jnp.float32),
            jax.ShapeDtypeStruct((n_rows_total, width), jnp.float32),
        ],
        scratch_types=(
            [pltpu.VMEM((ch,), jnp.int32)] * 2
            + [pltpu.VMEM((ch, width), jnp.float32)] * 4
            + [pltpu.SemaphoreType.DMA] * 8
        ),
    )
    def gather_k(ft_hbm, ct_hbm, idx_hbm, outf_hbm, outc_hbm,
                 i0, i1, f0, f1, c0, c1,
                 sgf0, sgf1, sgc0, sgc1, swf0, swf1, swc0, swc1):
        wid = lax.axis_index("s") * info.num_cores + lax.axis_index("c")
        idxb, fb, cb = [i0, i1], [f0, f1], [c0, c1]
        sgf, sgc = [sgf0, sgf1], [sgc0, sgc1]
        swf, swc = [swf0, swf1], [swc0, swc1]
        gops = [None, None]
        wops = [None, None]
        for c in range(n_ch):
            s = c % 2
            if wops[s] is not None:
                for op in wops[s]:
                    op.wait()
                wops[s] = None
            base = wid * per_w + c * ch
            pltpu.sync_copy(idx_hbm.at[pl.ds(base, ch)], idxb[s])
            gops[s] = (
                pltpu.async_copy(ft_hbm.at[idxb[s]], fb[s], sgf[s]),
                pltpu.async_copy(ct_hbm.at[idxb[s]], cb[s], sgc[s]),
                base,
            )
            s1 = 1 - s
            if gops[s1] is not None:
                gf1, gc1, b1 = gops[s1]
                gf1.wait()
                gc1.wait()
                wops[s1] = (
                    pltpu.async_copy(fb[s1], outf_hbm.at[pl.ds(b1, ch)],
                                     swf[s1]),
                    pltpu.async_copy(cb[s1], outc_hbm.at[pl.ds(b1, ch)],
                                     swc[s1]),
                )
                gops[s1] = None
        s = (n_ch - 1) % 2
        gf, gc, b1 = gops[s]
        gf.wait()
        gc.wait()
        wops[s] = (
            pltpu.async_copy(fb[s], outf_hbm.at[pl.ds(b1, ch)], swf[s]),
            pltpu.async_copy(cb[s], outc_hbm.at[pl.ds(b1, ch)], swc[s]),
        )
        for s in (0, 1):
            if wops[s] is not None:
                for op in wops[s]:
                    op.wait()

    return gather_k


def _mlp_body(g_ref, rowsf_ref, rowsc_ref, dist_ref, w1_ref, b1_ref, w2_ref,
              b2_ref, gamma_ref, beta_ref, out_ref):
    mc = g_ref.shape[1]
    g = g_ref[0]
    feats = rowsf_ref[0]
    pc = rowsc_ref[0][:, 0:3].reshape(mc, _K, 3)
    w1 = w1_ref[...]
    dot = lambda a, b: lax.dot_general(
        a, b, (((1,), (0,)), ((), ())), preferred_element_type=jnp.float32)
    rel = (g[:, None, :] - pc).reshape(mc * _K, 3)
    pre = dot(rel, w1).reshape(mc, _K, 128) + b1_ref[...][None]
    h = 0.5 * pre * (1.0 + lax.erf(pre * np.float32(1.0 / np.sqrt(2.0))))
    kappa = (dot(h.reshape(mc * _K, 128), w2_ref[...]).reshape(mc, _K, 128)
             + b2_ref[...][None])
    dist = dist_ref[0]
    w = 1.0 / (dist + 1e-6)
    w = w / jnp.sum(w, axis=1, keepdims=True)
    msg = kappa * feats.reshape(mc, _K, 128) * w[:, :, None]
    out = jnp.sum(msg, axis=1)
    mu = jnp.mean(out, axis=1, keepdims=True)
    var = jnp.mean((out - mu) ** 2, axis=1, keepdims=True)
    out = (out - mu) / jnp.sqrt(var + 1e-5)
    out_ref[0] = out * gamma_ref[...] + beta_ref[...]


def _mlp_call(grid_coords, rowsf, rowsc, dist, w1, b1, w2, b2, gamma, beta):
    B, M, _ = grid_coords.shape
    grid = (B, M // _MLP_MC)
    return pl.pallas_call(
        _mlp_body,
        grid=grid,
        in_specs=[
            pl.BlockSpec((1, _MLP_MC, 3), lambda b, i: (b, i, 0)),
            pl.BlockSpec((1, _MLP_MC * _K, 128), lambda b, i: (b, i, 0)),
            pl.BlockSpec((1, _MLP_MC * _K, 128), lambda b, i: (b, i, 0)),
            pl.BlockSpec((1, _MLP_MC, _K), lambda b, i: (b, i, 0)),
            pl.BlockSpec((3, 128), lambda b, i: (0, 0)),
            pl.BlockSpec((1, 128), lambda b, i: (0, 0)),
            pl.BlockSpec((128, 128), lambda b, i: (0, 0)),
            pl.BlockSpec((1, 128), lambda b, i: (0, 0)),
            pl.BlockSpec((1, 128), lambda b, i: (0, 0)),
            pl.BlockSpec((1, 128), lambda b, i: (0, 0)),
        ],
        out_specs=pl.BlockSpec((1, _MLP_MC, 128), lambda b, i: (b, i, 0)),
        out_shape=jax.ShapeDtypeStruct((B, M, 128), jnp.float32),
    )(grid_coords, rowsf, rowsc, dist, w1, b1, w2, b2, gamma, beta)


_N_SLICES = 2


def kernel(grid_coords, point_coords, point_feats, W1, b1, W2, b2, gamma,
           beta):
    B, M, _ = grid_coords.shape
    N = point_coords.shape[1]
    D = point_feats.shape[2]
    pt_t = jnp.transpose(point_coords, (0, 2, 1))
    ft = point_feats.reshape(B * N, D)
    ct = jnp.concatenate(
        [point_coords, jnp.zeros((B, N, D - 3), jnp.float32)],
        axis=-1).reshape(B * N, D)
    ms = M // _N_SLICES
    gather = _make_gather(B * ms * _K, D)
    topk = [_topk_call(grid_coords[:, s * ms:(s + 1) * ms], pt_t)
            for s in range(_N_SLICES)]
    outs = []
    for s in range(_N_SLICES):
        dist, idxf = topk[s]
        rowsf, rowsc = gather(ft, ct, idxf.reshape(-1))
        outs.append(_mlp_call(
            grid_coords[:, s * ms:(s + 1) * ms],
            rowsf.reshape(B, ms * _K, D), rowsc.reshape(B, ms * _K, D),
            dist, W1, b1.reshape(1, D), W2, b2.reshape(1, D),
            gamma.reshape(1, D), beta.reshape(1, D)))
    return jnp.concatenate(outs, axis=1)
